# baseline (speedup 1.0000x reference)
import jax
import jax.numpy as jnp
from jax.experimental import pallas as pl

N = 10000
LAMBDA1 = 3.0
LAMBDA2 = 3.0
K = 5


def _mlp_body(x_ref, w1_ref, w2_ref, o_ref):
    h = jnp.maximum(x_ref[...] @ w1_ref[...].T, 0.0)
    o_ref[...] = h @ w2_ref[...].T


def kernel(x, edge_index, W1, W2):
    n, nfeat = x.shape
    nhid = W1.shape[0]
    ncls = W2.shape[0]
    blk = 1000
    h = pl.pallas_call(
        _mlp_body,
        grid=(n // blk,),
        in_specs=[
            pl.BlockSpec((blk, nfeat), lambda i: (i, 0)),
            pl.BlockSpec((nhid, nfeat), lambda i: (0, 0)),
            pl.BlockSpec((ncls, nhid), lambda i: (0, 0)),
        ],
        out_specs=pl.BlockSpec((blk, ncls), lambda i: (i, 0)),
        out_shape=jax.ShapeDtypeStruct((n, ncls), x.dtype),
    )(x, W1, W2)

    src = edge_index[0]
    dst = edge_index[1]
    dt = h.dtype
    m = (src != dst).astype(dt)
    rows = jnp.concatenate([src, dst])
    cols = jnp.concatenate([dst, src])
    w = jnp.concatenate([m, m])
    deg = jnp.zeros((n,), dt).at[rows].add(w) + 1.0
    dinv = deg ** -0.5
    wn = w * dinv[rows] * dinv[cols]
    dself = dinv * dinv

    def prop(v):
        return jnp.zeros_like(v).at[rows].add(wn[:, None] * v[cols]) + dself[:, None] * v

    hi = jnp.maximum(src, dst)
    lo = jnp.minimum(src, dst)
    chi = m * dinv[hi]
    clo = m * dinv[lo]

    def inc_mv(v):
        return chi[:, None] * v[hi] - clo[:, None] * v[lo]

    def inc_t_mv(z):
        out = jnp.zeros((n, z.shape[1]), z.dtype)
        out = out.at[hi].add(chi[:, None] * z)
        out = out.at[lo].add(-clo[:, None] * z)
        return out

    gamma = 1.0 / (1.0 + LAMBDA2)
    beta = 1.0 / (2.0 * gamma)
    hh = h
    xx = h
    z = jnp.zeros((edge_index.shape[1], h.shape[1]), dt)
    for _ in range(K):
        y = gamma * hh + (1.0 - gamma) * prop(xx)
        x_bar = y - gamma * inc_t_mv(z)
        z_bar = z + beta * inc_mv(x_bar)
        rn = jnp.linalg.norm(z_bar, axis=1)
        scale = jnp.minimum(rn, LAMBDA1)
        scale = jnp.where(rn > 0, scale / jnp.where(rn > 0, rn, 1.0), scale)
        z = scale[:, None] * z_bar
        xx = y - gamma * inc_t_mv(z)
    return jax.nn.log_softmax(xx, axis=1)


# speedup vs baseline: 14.2965x; 14.2965x over previous
import functools

import jax
import jax.numpy as jnp
from jax import lax
from jax.experimental import pallas as pl
from jax.experimental.pallas import tpu as pltpu
from jax.experimental.pallas import tpu_sc as plsc

L1 = 3.0
L2 = 3.0
KITER = 5
GAMMA = 1.0 / (1.0 + L2)
BETA = 1.0 / (2.0 * GAMMA)

NC = 2
NS = 16
NW = NC * NS
CH = 128


def _rsqrt16(ss):
    i = lax.bitcast_convert_type(ss, jnp.int32)
    y = lax.bitcast_convert_type(
        jnp.full((16,), 0x5F3759DF, jnp.int32) - (i >> 1), jnp.float32)
    for _ in range(3):
        y = y * (1.5 - 0.5 * ss * y * y)
    return y


def _mlp_body(x_ref, w1_ref, w2_ref, o_ref):
    h = jnp.maximum(x_ref[...] @ w1_ref[...].T, 0.0)
    o_ref[...] = h @ w2_ref[...].T


def _fin_body(y_ref, t0_ref, t1_ref, o_ref):
    xx = y_ref[...] - GAMMA * (t0_ref[...] + t1_ref[...])
    m = jnp.max(xx, axis=1, keepdims=True)
    e = jnp.exp(xx - m)
    o_ref[...] = (xx - m) - jnp.log(jnp.sum(e, axis=1, keepdims=True))


def kernel(x, edge_index, W1, W2):
    n, nfeat = x.shape
    nhid = W1.shape[0]
    ncls = W2.shape[0]
    e = edge_index.shape[1]
    src = edge_index[0].astype(jnp.int32)
    dst = edge_index[1].astype(jnp.int32)

    NP = -(-n // 512) * 512
    TWO_E = 2 * e
    PP = -(-(TWO_E + n) // (NW * CH)) * (NW * CH)
    EP = -(-e // (NW * CH)) * (NW * CH)
    PW = PP // NW
    NCHA = PW // CH
    EW = EP // NW
    NCHB = EW // CH
    NPS = NP // NS
    RW = NP // NW

    mesh = plsc.VectorSubcoreMesh(core_axis_name="c", subcore_axis_name="s")
    f32 = jnp.float32
    i32 = jnp.int32

    diag = jnp.arange(n, dtype=i32)
    padp = jnp.zeros((PP - TWO_E - n,), i32)
    pr3 = jnp.concatenate([src, dst, diag, padp]).reshape(NW, NCHA, CH)
    pc3 = jnp.concatenate([dst, src, diag, padp]).reshape(NW, NCHA, CH)
    pade = jnp.zeros((EP - e,), i32)
    se3 = jnp.concatenate([src, pade]).reshape(NW, NCHB, CH)
    de3 = jnp.concatenate([dst, pade]).reshape(NW, NCHB, CH)
    zflat = jnp.zeros((NP,), f32)
    z16 = jnp.zeros((NP, ncls), f32)
    z0 = jnp.zeros((NW, NCHB, CH // 16, 16, 16), f32)

    blk = 1000
    h = pl.pallas_call(
        _mlp_body,
        grid=(n // blk,),
        in_specs=[
            pl.BlockSpec((blk, nfeat), lambda i: (i, 0)),
            pl.BlockSpec((nhid, nfeat), lambda i: (0, 0)),
            pl.BlockSpec((ncls, nhid), lambda i: (0, 0)),
        ],
        out_specs=pl.BlockSpec((blk, ncls), lambda i: (i, 0)),
        out_shape=jax.ShapeDtypeStruct((n, ncls), f32),
    )(x, W1, W2)
    hp = jnp.pad(h, ((0, NP - n), (0, 0)))

    iot = lambda: lax.iota(i32, 16)

    def _deg_body(pr_hbm, pc_hbm, zf_hbm, deg_hbm, rv, cv, wv, ridx, accs):
        c = lax.axis_index("c")
        s = lax.axis_index("s")
        w = c * NS + s
        pltpu.sync_copy(zf_hbm.at[pl.ds(s * NPS, NPS)],
                        accs.at[pl.ds(s * NPS, NPS)])
        pltpu.sync_copy(pr_hbm.at[w], rv)
        pltpu.sync_copy(pc_hbm.at[w], cv)
        plsc.subcore_barrier()

        def chunk(j, carry):
            for g in range(CH // 16):
                r16 = rv[j, pl.ds(g * 16, 16)]
                c16 = cv[j, pl.ds(g * 16, 16)]
                pos = w * PW + j * CH + g * 16 + iot()
                valid = (r16 != c16) | ((pos >= TWO_E) & (pos < TWO_E + n))
                wv[pl.ds(g * 16, 16)] = jnp.where(valid, 1.0, 0.0).astype(f32)
                ridx[pl.ds(g * 16, 16)] = r16
            pltpu.sync_copy(wv, accs.at[ridx], add=True)
            return carry

        lax.fori_loop(0, NCHA, chunk, 0)
        plsc.subcore_barrier()
        pltpu.sync_copy(accs.at[pl.ds(s * NPS, NPS)],
                        deg_hbm.at[pl.ds(c * NP + s * NPS, NPS)])

    deg_part = pl.kernel(
        _deg_body,
        out_type=jax.ShapeDtypeStruct((NC * NP,), f32),
        mesh=mesh,
        compiler_params=pltpu.CompilerParams(needs_layout_passes=False, use_tc_tiling_on_sc=False),
        scratch_types=[
            pltpu.VMEM((NCHA, CH), i32),
            pltpu.VMEM((NCHA, CH), i32),
            pltpu.VMEM((CH,), f32),
            pltpu.VMEM((CH,), i32),
            pltpu.VMEM_SHARED((NP,), f32),
        ],
    )(pr3, pc3, zflat)

    def _dinv_body(deg_hbm, dinv_hbm, d0, d1, db):
        c = lax.axis_index("c")
        s = lax.axis_index("s")
        w = c * NS + s
        pltpu.sync_copy(deg_hbm.at[pl.ds(w * RW, RW)], d0)
        pltpu.sync_copy(deg_hbm.at[pl.ds(NP + w * RW, RW)], d1)

        def vstep(i, carry):
            ss = d0[pl.ds(i * 16, 16)] + d1[pl.ds(i * 16, 16)]
            db[pl.ds(i * 16, 16)] = _rsqrt16(ss)
            return carry

        lax.fori_loop(0, RW // 16, vstep, 0)
        pltpu.sync_copy(db, dinv_hbm.at[pl.ds(w * RW, RW)])

    dinv = pl.kernel(
        _dinv_body,
        out_type=jax.ShapeDtypeStruct((NP,), f32),
        mesh=mesh,
        compiler_params=pltpu.CompilerParams(needs_layout_passes=False, use_tc_tiling_on_sc=False),
        scratch_types=[
            pltpu.VMEM((RW,), f32),
            pltpu.VMEM((RW,), f32),
            pltpu.VMEM((RW,), f32),
        ],
    )(deg_part)

    def _propa_body(pr_hbm, pc_hbm, dinv_hbm, xx_hbm, z16_hbm, accp_hbm,
                    rv, cv, dv, gbuf, cbuf, ridx, accs, sem):
        c = lax.axis_index("c")
        s = lax.axis_index("s")
        w = c * NS + s
        pltpu.sync_copy(z16_hbm.at[pl.ds(s * NPS, NPS)],
                        accs.at[pl.ds(s * NPS, NPS)])
        pltpu.sync_copy(pr_hbm.at[w], rv)
        pltpu.sync_copy(pc_hbm.at[w], cv)
        pltpu.sync_copy(dinv_hbm, dv)
        plsc.subcore_barrier()

        def chunk(j, carry):
            pltpu.async_copy(xx_hbm.at[cv.at[j]], gbuf, sem).wait()
            for g in range(CH // 16):
                r16 = rv[j, pl.ds(g * 16, 16)]
                c16 = cv[j, pl.ds(g * 16, 16)]
                pos = w * PW + j * CH + g * 16 + iot()
                dr = plsc.load_gather(dv, [r16])
                dc = plsc.load_gather(dv, [c16])
                valid = (r16 != c16) | ((pos >= TWO_E) & (pos < TWO_E + n))
                wt = jnp.where(valid, dr * dc, 0.0)
                ridx[pl.ds(g * 16, 16)] = r16
                rows = g * 16 + iot()
                for f in range(16):
                    fv = jnp.full((16,), f, i32)
                    colf = plsc.load_gather(gbuf, [rows, fv])
                    plsc.store_scatter(cbuf, [rows, fv], wt * colf)
            pltpu.sync_copy(cbuf, accs.at[ridx], add=True)
            return carry

        lax.fori_loop(0, NCHA, chunk, 0)
        plsc.subcore_barrier()
        pltpu.sync_copy(accs.at[pl.ds(s * NPS, NPS)],
                        accp_hbm.at[c, pl.ds(s * NPS, NPS)])

    propa = pl.kernel(
        _propa_body,
        out_type=jax.ShapeDtypeStruct((NC, NP, ncls), f32),
        mesh=mesh,
        compiler_params=pltpu.CompilerParams(needs_layout_passes=False, use_tc_tiling_on_sc=False),
        scratch_types=[
            pltpu.VMEM((NCHA, CH), i32),
            pltpu.VMEM((NCHA, CH), i32),
            pltpu.VMEM((NP,), f32),
            pltpu.VMEM((CH, ncls), f32),
            pltpu.VMEM((CH, ncls), f32),
            pltpu.VMEM((CH,), i32),
            pltpu.VMEM_SHARED((NP, ncls), f32),
            pltpu.SemaphoreType.DMA,
        ],
    )

    def _n1_body(h_hbm, p_hbm, t_hbm, y_hbm, xb_hbm, hb, p0b, p1b, tb, yb, xbb):
        c = lax.axis_index("c")
        s = lax.axis_index("s")
        w = c * NS + s
        pltpu.sync_copy(h_hbm.at[pl.ds(w * RW, RW)], hb)
        pltpu.sync_copy(p_hbm.at[0, pl.ds(w * RW, RW)], p0b)
        pltpu.sync_copy(p_hbm.at[1, pl.ds(w * RW, RW)], p1b)
        pltpu.sync_copy(t_hbm.at[pl.ds(w * RW, RW)], tb)

        def row(r, carry):
            yv = GAMMA * hb[r, :] + (1.0 - GAMMA) * (p0b[r, :] + p1b[r, :])
            yb[r, :] = yv
            xbb[r, :] = yv - GAMMA * tb[r, :]
            return carry

        lax.fori_loop(0, RW, row, 0)
        pltpu.sync_copy(yb, y_hbm.at[pl.ds(w * RW, RW)])
        pltpu.sync_copy(xbb, xb_hbm.at[pl.ds(w * RW, RW)])

    n1k = pl.kernel(
        _n1_body,
        out_type=(jax.ShapeDtypeStruct((NP, ncls), f32),
                  jax.ShapeDtypeStruct((NP, ncls), f32)),
        mesh=mesh,
        compiler_params=pltpu.CompilerParams(needs_layout_passes=False, use_tc_tiling_on_sc=False),
        scratch_types=[pltpu.VMEM((RW, ncls), f32) for _ in range(6)],
    )

    def _passb_body(se_hbm, de_hbm, dinv_hbm, xb_hbm, zin_hbm, z16_hbm,
                    zout_hbm, acct_hbm,
                    sv, dvv, dv, ga, gb, zc, zo, chb, clb, hidx, lidx,
                    accs, sem):
        c = lax.axis_index("c")
        s = lax.axis_index("s")
        w = c * NS + s
        pltpu.sync_copy(z16_hbm.at[pl.ds(s * NPS, NPS)],
                        accs.at[pl.ds(s * NPS, NPS)])
        pltpu.sync_copy(se_hbm.at[w], sv)
        pltpu.sync_copy(de_hbm.at[w], dvv)
        pltpu.sync_copy(dinv_hbm, dv)
        plsc.subcore_barrier()

        def chunk(j, carry):
            for g in range(CH // 16):
                s16 = sv[j, pl.ds(g * 16, 16)]
                d16 = dvv[j, pl.ds(g * 16, 16)]
                hidx[pl.ds(g * 16, 16)] = jnp.maximum(s16, d16)
                lidx[pl.ds(g * 16, 16)] = jnp.minimum(s16, d16)
            pltpu.async_copy(xb_hbm.at[hidx], ga, sem).wait()
            pltpu.async_copy(xb_hbm.at[lidx], gb, sem).wait()
            pltpu.sync_copy(zin_hbm.at[w, j], zc)
            for g in range(CH // 16):
                s16 = sv[j, pl.ds(g * 16, 16)]
                d16 = dvv[j, pl.ds(g * 16, 16)]
                hi16 = hidx[pl.ds(g * 16, 16)]
                lo16 = lidx[pl.ds(g * 16, 16)]
                m = s16 != d16
                chi = jnp.where(m, plsc.load_gather(dv, [hi16]), 0.0)
                clo = jnp.where(m, plsc.load_gather(dv, [lo16]), 0.0)
                rows = g * 16 + iot()
                ss = jnp.zeros((16,), f32)
                zb = []
                for f in range(16):
                    fv = jnp.full((16,), f, i32)
                    xaf = plsc.load_gather(ga, [rows, fv])
                    xbf = plsc.load_gather(gb, [rows, fv])
                    vf = chi * xaf - clo * xbf
                    zbf = zc[g, f, :] + BETA * vf
                    ss = ss + zbf * zbf
                    zb.append(zbf)
                scale = jnp.minimum(1.0, L1 * _rsqrt16(ss))
                for f in range(16):
                    fv = jnp.full((16,), f, i32)
                    zsc = scale * zb[f]
                    zo[g, f, :] = zsc
                    plsc.store_scatter(chb, [rows, fv], chi * zsc)
                    plsc.store_scatter(clb, [rows, fv], -clo * zsc)
            pltpu.sync_copy(zo, zout_hbm.at[w, j])
            pltpu.sync_copy(chb, accs.at[hidx], add=True)
            pltpu.sync_copy(clb, accs.at[lidx], add=True)
            return carry

        lax.fori_loop(0, NCHB, chunk, 0)
        plsc.subcore_barrier()
        pltpu.sync_copy(accs.at[pl.ds(s * NPS, NPS)],
                        acct_hbm.at[c, pl.ds(s * NPS, NPS)])

    passb = pl.kernel(
        _passb_body,
        out_type=(jax.ShapeDtypeStruct((NW, NCHB, CH // 16, 16, 16), f32),
                  jax.ShapeDtypeStruct((NC, NP, ncls), f32)),
        mesh=mesh,
        compiler_params=pltpu.CompilerParams(needs_layout_passes=False, use_tc_tiling_on_sc=False),
        scratch_types=[
            pltpu.VMEM((NCHB, CH), i32),
            pltpu.VMEM((NCHB, CH), i32),
            pltpu.VMEM((NP,), f32),
            pltpu.VMEM((CH, ncls), f32),
            pltpu.VMEM((CH, ncls), f32),
            pltpu.VMEM((CH // 16, 16, 16), f32),
            pltpu.VMEM((CH // 16, 16, 16), f32),
            pltpu.VMEM((CH, ncls), f32),
            pltpu.VMEM((CH, ncls), f32),
            pltpu.VMEM((CH,), i32),
            pltpu.VMEM((CH,), i32),
            pltpu.VMEM_SHARED((NP, ncls), f32),
            pltpu.SemaphoreType.DMA,
        ],
    )

    def _n2_body(y_hbm, t_hbm, xx_hbm, tc_hbm, yb, t0b, t1b, xxb, tcb):
        c = lax.axis_index("c")
        s = lax.axis_index("s")
        w = c * NS + s
        pltpu.sync_copy(y_hbm.at[pl.ds(w * RW, RW)], yb)
        pltpu.sync_copy(t_hbm.at[0, pl.ds(w * RW, RW)], t0b)
        pltpu.sync_copy(t_hbm.at[1, pl.ds(w * RW, RW)], t1b)

        def row(r, carry):
            tv = t0b[r, :] + t1b[r, :]
            tcb[r, :] = tv
            xxb[r, :] = yb[r, :] - GAMMA * tv
            return carry

        lax.fori_loop(0, RW, row, 0)
        pltpu.sync_copy(xxb, xx_hbm.at[pl.ds(w * RW, RW)])
        pltpu.sync_copy(tcb, tc_hbm.at[pl.ds(w * RW, RW)])

    n2k = pl.kernel(
        _n2_body,
        out_type=(jax.ShapeDtypeStruct((NP, ncls), f32),
                  jax.ShapeDtypeStruct((NP, ncls), f32)),
        mesh=mesh,
        compiler_params=pltpu.CompilerParams(needs_layout_passes=False, use_tc_tiling_on_sc=False),
        scratch_types=[pltpu.VMEM((RW, ncls), f32) for _ in range(5)],
    )

    xx = hp
    tcomb = z16
    zcur = z0
    y = hp
    acct = None
    for it in range(KITER):
        accp = propa(pr3, pc3, dinv, xx, z16)
        y, xbar = n1k(hp, accp, tcomb)
        zcur, acct = passb(se3, de3, dinv, xbar, zcur, z16)
        if it < KITER - 1:
            xx, tcomb = n2k(y, acct)

    fblk = 1280
    out = pl.pallas_call(
        _fin_body,
        grid=(NP // fblk,),
        in_specs=[
            pl.BlockSpec((fblk, ncls), lambda i: (i, 0)),
            pl.BlockSpec((fblk, ncls), lambda i: (i, 0)),
            pl.BlockSpec((fblk, ncls), lambda i: (i, 0)),
        ],
        out_specs=pl.BlockSpec((fblk, ncls), lambda i: (i, 0)),
        out_shape=jax.ShapeDtypeStruct((NP, ncls), f32),
    )(y, acct[0], acct[1])
    return out[:n]


# speedup vs baseline: 22.0913x; 1.5452x over previous
import functools

import jax
import jax.numpy as jnp
from jax import lax
from jax.experimental import pallas as pl
from jax.experimental.pallas import tpu as pltpu
from jax.experimental.pallas import tpu_sc as plsc

L1 = 3.0
L2 = 3.0
KITER = 5
GAMMA = 1.0 / (1.0 + L2)
BETA = 1.0 / (2.0 * GAMMA)

NC = 2
NS = 16
NW = NC * NS
CH = 128


def _rsqrt16(ss):
    i = lax.bitcast_convert_type(ss, jnp.int32)
    y = lax.bitcast_convert_type(
        jnp.full((16,), 0x5F3759DF, jnp.int32) - (i >> 1), jnp.float32)
    for _ in range(3):
        y = y * (1.5 - 0.5 * ss * y * y)
    return y


def _mlp_body(x_ref, w1_ref, w2_ref, o_ref):
    h = jnp.maximum(x_ref[...] @ w1_ref[...].T, 0.0)
    o_ref[...] = h @ w2_ref[...].T


def _fin_body(y_ref, t0_ref, t1_ref, o_ref):
    xx = y_ref[...] - GAMMA * (t0_ref[...] + t1_ref[...])
    m = jnp.max(xx, axis=1, keepdims=True)
    e = jnp.exp(xx - m)
    o_ref[...] = (xx - m) - jnp.log(jnp.sum(e, axis=1, keepdims=True))


def kernel(x, edge_index, W1, W2):
    n, nfeat = x.shape
    nhid = W1.shape[0]
    ncls = W2.shape[0]
    e = edge_index.shape[1]
    src = edge_index[0].astype(jnp.int32)
    dst = edge_index[1].astype(jnp.int32)

    NP = -(-n // 512) * 512
    TWO_E = 2 * e
    PP = -(-(TWO_E + n) // (NW * CH)) * (NW * CH)
    EP = -(-e // (NW * CH)) * (NW * CH)
    PW = PP // NW
    NCHA = PW // CH
    EW = EP // NW
    NCHB = EW // CH
    NPS = NP // NS
    RW = NP // NW

    mesh = plsc.VectorSubcoreMesh(core_axis_name="c", subcore_axis_name="s")
    f32 = jnp.float32
    i32 = jnp.int32

    diag = jnp.arange(n, dtype=i32)
    padp = jnp.zeros((PP - TWO_E - n,), i32)
    pr3 = jnp.concatenate([src, dst, diag, padp]).reshape(NW, NCHA, CH)
    pc3 = jnp.concatenate([dst, src, diag, padp]).reshape(NW, NCHA, CH)
    pade = jnp.zeros((EP - e,), i32)
    se3 = jnp.concatenate([src, pade]).reshape(NW, NCHB, CH)
    de3 = jnp.concatenate([dst, pade]).reshape(NW, NCHB, CH)
    zflat = jnp.zeros((NP,), f32)
    z16 = jnp.zeros((NP, ncls), f32)
    z0 = jnp.zeros((NW, NCHB, CH // 16, 16, 16), f32)

    blk = 1000
    h = pl.pallas_call(
        _mlp_body,
        grid=(n // blk,),
        in_specs=[
            pl.BlockSpec((blk, nfeat), lambda i: (i, 0)),
            pl.BlockSpec((nhid, nfeat), lambda i: (0, 0)),
            pl.BlockSpec((ncls, nhid), lambda i: (0, 0)),
        ],
        out_specs=pl.BlockSpec((blk, ncls), lambda i: (i, 0)),
        out_shape=jax.ShapeDtypeStruct((n, ncls), f32),
    )(x, W1, W2)
    hp = jnp.pad(h, ((0, NP - n), (0, 0)))

    iot = lambda: lax.iota(i32, 16)

    def _deg_body(pr_hbm, pc_hbm, zf_hbm, deg_hbm, rv, cv, wv, ridx, accs):
        c = lax.axis_index("c")
        s = lax.axis_index("s")
        w = c * NS + s
        pltpu.sync_copy(zf_hbm.at[pl.ds(s * NPS, NPS)],
                        accs.at[pl.ds(s * NPS, NPS)])
        pltpu.sync_copy(pr_hbm.at[w], rv)
        pltpu.sync_copy(pc_hbm.at[w], cv)
        plsc.subcore_barrier()

        def chunk(j, carry):
            for g in range(CH // 16):
                r16 = rv[j, pl.ds(g * 16, 16)]
                c16 = cv[j, pl.ds(g * 16, 16)]
                pos = w * PW + j * CH + g * 16 + iot()
                valid = (r16 != c16) | ((pos >= TWO_E) & (pos < TWO_E + n))
                wv[pl.ds(g * 16, 16)] = jnp.where(valid, 1.0, 0.0).astype(f32)
                ridx[pl.ds(g * 16, 16)] = r16
            pltpu.sync_copy(wv, accs.at[ridx], add=True)
            return carry

        lax.fori_loop(0, NCHA, chunk, 0)
        plsc.subcore_barrier()
        pltpu.sync_copy(accs.at[pl.ds(s * NPS, NPS)],
                        deg_hbm.at[pl.ds(c * NP + s * NPS, NPS)])

    deg_part = pl.kernel(
        _deg_body,
        out_type=jax.ShapeDtypeStruct((NC * NP,), f32),
        mesh=mesh,
        compiler_params=pltpu.CompilerParams(needs_layout_passes=False, use_tc_tiling_on_sc=False),
        scratch_types=[
            pltpu.VMEM((NCHA, CH), i32),
            pltpu.VMEM((NCHA, CH), i32),
            pltpu.VMEM((CH,), f32),
            pltpu.VMEM((CH,), i32),
            pltpu.VMEM_SHARED((NP,), f32),
        ],
    )(pr3, pc3, zflat)

    def _dinv_body(deg_hbm, dinv_hbm, d0, d1, db):
        c = lax.axis_index("c")
        s = lax.axis_index("s")
        w = c * NS + s
        pltpu.sync_copy(deg_hbm.at[pl.ds(w * RW, RW)], d0)
        pltpu.sync_copy(deg_hbm.at[pl.ds(NP + w * RW, RW)], d1)

        def vstep(i, carry):
            ss = d0[pl.ds(i * 16, 16)] + d1[pl.ds(i * 16, 16)]
            db[pl.ds(i * 16, 16)] = _rsqrt16(ss)
            return carry

        lax.fori_loop(0, RW // 16, vstep, 0)
        pltpu.sync_copy(db, dinv_hbm.at[pl.ds(w * RW, RW)])

    dinv = pl.kernel(
        _dinv_body,
        out_type=jax.ShapeDtypeStruct((NP,), f32),
        mesh=mesh,
        compiler_params=pltpu.CompilerParams(needs_layout_passes=False, use_tc_tiling_on_sc=False),
        scratch_types=[
            pltpu.VMEM((RW,), f32),
            pltpu.VMEM((RW,), f32),
            pltpu.VMEM((RW,), f32),
        ],
    )(deg_part)

    def _propa_body(pr_hbm, pc_hbm, dinv_hbm, xx_hbm, z16_hbm, accp_hbm,
                    rv, cv, dv, gbuf0, gbuf1, cbuf0, cbuf1, ridx0, ridx1,
                    accs, gsem0, gsem1, ssem0, ssem1):
        c = lax.axis_index("c")
        s = lax.axis_index("s")
        w = c * NS + s
        gbuf = (gbuf0, gbuf1)
        cbuf = (cbuf0, cbuf1)
        ridx = (ridx0, ridx1)
        gsem = (gsem0, gsem1)
        ssem = (ssem0, ssem1)
        pltpu.sync_copy(z16_hbm.at[pl.ds(s * NPS, NPS)],
                        accs.at[pl.ds(s * NPS, NPS)])
        pltpu.sync_copy(pr_hbm.at[w], rv)
        pltpu.sync_copy(pc_hbm.at[w], cv)
        pltpu.sync_copy(dinv_hbm, dv)
        plsc.subcore_barrier()
        pltpu.async_copy(xx_hbm.at[cv.at[0]], gbuf[0], gsem[0])

        def compute(j, p):
            for g in range(CH // 16):
                r16 = rv[j, pl.ds(g * 16, 16)]
                c16 = cv[j, pl.ds(g * 16, 16)]
                pos = w * PW + j * CH + g * 16 + iot()
                dr = plsc.load_gather(dv, [r16])
                dc = plsc.load_gather(dv, [c16])
                valid = (r16 != c16) | ((pos >= TWO_E) & (pos < TWO_E + n))
                wt = jnp.where(valid, dr * dc, 0.0)
                ridx[p][pl.ds(g * 16, 16)] = r16
                rows = g * 16 + iot()
                for f in range(16):
                    fv = jnp.full((16,), f, i32)
                    colf = plsc.load_gather(gbuf[p], [rows, fv])
                    plsc.store_scatter(cbuf[p], [rows, fv], wt * colf)

        NJJ = NCHA // 2

        def step(jj, carry):
            for p in (0, 1):
                j = 2 * jj + p
                if p == 0:
                    pltpu.async_copy(xx_hbm.at[cv.at[j + 1]], gbuf[1], gsem[1])
                else:
                    @pl.when(jj < NJJ - 1)
                    def _():
                        pltpu.async_copy(xx_hbm.at[cv.at[j + 1]], gbuf[0],
                                         gsem[0])
                pltpu.make_async_copy(xx_hbm.at[cv.at[j]], gbuf[p],
                                      gsem[p]).wait()
                compute(j, p)
                pltpu.sync_copy(cbuf[p], accs.at[ridx[p]], add=True)
            return carry

        lax.fori_loop(0, NJJ, step, 0)
        plsc.subcore_barrier()
        pltpu.sync_copy(accs.at[pl.ds(s * NPS, NPS)],
                        accp_hbm.at[c, pl.ds(s * NPS, NPS)])

    propa = pl.kernel(
        _propa_body,
        out_type=jax.ShapeDtypeStruct((NC, NP, ncls), f32),
        mesh=mesh,
        compiler_params=pltpu.CompilerParams(needs_layout_passes=False, use_tc_tiling_on_sc=False),
        scratch_types=[
            pltpu.VMEM((NCHA, CH), i32),
            pltpu.VMEM((NCHA, CH), i32),
            pltpu.VMEM((NP,), f32),
            pltpu.VMEM((CH, ncls), f32),
            pltpu.VMEM((CH, ncls), f32),
            pltpu.VMEM((CH, ncls), f32),
            pltpu.VMEM((CH, ncls), f32),
            pltpu.VMEM((CH,), i32),
            pltpu.VMEM((CH,), i32),
            pltpu.VMEM_SHARED((NP, ncls), f32),
            pltpu.SemaphoreType.DMA,
            pltpu.SemaphoreType.DMA,
            pltpu.SemaphoreType.DMA,
            pltpu.SemaphoreType.DMA,
        ],
    )

    def _n1_body(h_hbm, p_hbm, t_hbm, y_hbm, xb_hbm, hb, p0b, p1b, tb, yb, xbb):
        c = lax.axis_index("c")
        s = lax.axis_index("s")
        w = c * NS + s
        pltpu.sync_copy(h_hbm.at[pl.ds(w * RW, RW)], hb)
        pltpu.sync_copy(p_hbm.at[0, pl.ds(w * RW, RW)], p0b)
        pltpu.sync_copy(p_hbm.at[1, pl.ds(w * RW, RW)], p1b)
        pltpu.sync_copy(t_hbm.at[pl.ds(w * RW, RW)], tb)

        def row(r, carry):
            yv = GAMMA * hb[r, :] + (1.0 - GAMMA) * (p0b[r, :] + p1b[r, :])
            yb[r, :] = yv
            xbb[r, :] = yv - GAMMA * tb[r, :]
            return carry

        lax.fori_loop(0, RW, row, 0)
        pltpu.sync_copy(yb, y_hbm.at[pl.ds(w * RW, RW)])
        pltpu.sync_copy(xbb, xb_hbm.at[pl.ds(w * RW, RW)])

    n1k = pl.kernel(
        _n1_body,
        out_type=(jax.ShapeDtypeStruct((NP, ncls), f32),
                  jax.ShapeDtypeStruct((NP, ncls), f32)),
        mesh=mesh,
        compiler_params=pltpu.CompilerParams(needs_layout_passes=False, use_tc_tiling_on_sc=False),
        scratch_types=[pltpu.VMEM((RW, ncls), f32) for _ in range(6)],
    )

    def _passb_body(se_hbm, de_hbm, dinv_hbm, xb_hbm, zin_hbm, z16_hbm,
                    zout_hbm, acct_hbm,
                    sv, dvv, dv,
                    ga0, ga1, gb0, gb1, zc0, zc1, zo0, zo1,
                    chb0, chb1, clb0, clb1,
                    hidx0, hidx1, lidx0, lidx1,
                    shidx0, shidx1, slidx0, slidx1,
                    accs, gsem0, gsem1, ssem0, ssem1):
        c = lax.axis_index("c")
        s = lax.axis_index("s")
        w = c * NS + s
        ga = (ga0, ga1)
        gb = (gb0, gb1)
        zc = (zc0, zc1)
        zo = (zo0, zo1)
        chb = (chb0, chb1)
        clb = (clb0, clb1)
        hidx = (hidx0, hidx1)
        lidx = (lidx0, lidx1)
        shidx = (shidx0, shidx1)
        slidx = (slidx0, slidx1)
        gsem = (gsem0, gsem1)
        ssem = (ssem0, ssem1)
        pltpu.sync_copy(z16_hbm.at[pl.ds(s * NPS, NPS)],
                        accs.at[pl.ds(s * NPS, NPS)])
        pltpu.sync_copy(se_hbm.at[w], sv)
        pltpu.sync_copy(de_hbm.at[w], dvv)
        pltpu.sync_copy(dinv_hbm, dv)
        plsc.subcore_barrier()

        def build_idx(j, p):
            for g in range(CH // 16):
                s16 = sv[j, pl.ds(g * 16, 16)]
                d16 = dvv[j, pl.ds(g * 16, 16)]
                hidx[p][pl.ds(g * 16, 16)] = jnp.maximum(s16, d16)
                lidx[p][pl.ds(g * 16, 16)] = jnp.minimum(s16, d16)

        def issue_gathers(j, p):
            pltpu.async_copy(xb_hbm.at[hidx[p]], ga[p], gsem[p])
            pltpu.async_copy(xb_hbm.at[lidx[p]], gb[p], gsem[p])
            pltpu.async_copy(zin_hbm.at[w, j], zc[p], gsem[p])

        def wait_gathers(j, p):
            pltpu.make_async_copy(xb_hbm.at[hidx[p]], ga[p], gsem[p]).wait()
            pltpu.make_async_copy(xb_hbm.at[lidx[p]], gb[p], gsem[p]).wait()
            pltpu.make_async_copy(zin_hbm.at[w, j], zc[p], gsem[p]).wait()

        def compute(j, p):
            for g in range(CH // 16):
                s16 = sv[j, pl.ds(g * 16, 16)]
                d16 = dvv[j, pl.ds(g * 16, 16)]
                hi16 = hidx[p][pl.ds(g * 16, 16)]
                lo16 = lidx[p][pl.ds(g * 16, 16)]
                shidx[p][pl.ds(g * 16, 16)] = hi16
                slidx[p][pl.ds(g * 16, 16)] = lo16
                m = s16 != d16
                chi = jnp.where(m, plsc.load_gather(dv, [hi16]), 0.0)
                clo = jnp.where(m, plsc.load_gather(dv, [lo16]), 0.0)
                rows = g * 16 + iot()
                ss = jnp.zeros((16,), f32)
                zb = []
                for f in range(16):
                    fv = jnp.full((16,), f, i32)
                    xaf = plsc.load_gather(ga[p], [rows, fv])
                    xbf = plsc.load_gather(gb[p], [rows, fv])
                    vf = chi * xaf - clo * xbf
                    zbf = zc[p][g, f, :] + BETA * vf
                    ss = ss + zbf * zbf
                    zb.append(zbf)
                scale = jnp.minimum(1.0, L1 * _rsqrt16(ss))
                for f in range(16):
                    fv = jnp.full((16,), f, i32)
                    zsc = scale * zb[f]
                    zo[p][g, f, :] = zsc
                    plsc.store_scatter(chb[p], [rows, fv], chi * zsc)
                    plsc.store_scatter(clb[p], [rows, fv], -clo * zsc)

        build_idx(0, 0)
        issue_gathers(0, 0)
        NJJ = NCHB // 2

        def step(jj, carry):
            for p in (0, 1):
                j = 2 * jj + p
                if p == 0:
                    build_idx(j + 1, 1)
                    issue_gathers(j + 1, 1)
                else:
                    @pl.when(jj < NJJ - 1)
                    def _():
                        build_idx(j + 1, 0)
                        issue_gathers(j + 1, 0)
                wait_gathers(j, p)
                compute(j, p)
                pltpu.sync_copy(zo[p], zout_hbm.at[w, j])
                pltpu.sync_copy(chb[p], accs.at[shidx[p]], add=True)
                pltpu.sync_copy(clb[p], accs.at[slidx[p]], add=True)
            return carry

        lax.fori_loop(0, NJJ, step, 0)
        plsc.subcore_barrier()
        pltpu.sync_copy(accs.at[pl.ds(s * NPS, NPS)],
                        acct_hbm.at[c, pl.ds(s * NPS, NPS)])

    passb = pl.kernel(
        _passb_body,
        out_type=(jax.ShapeDtypeStruct((NW, NCHB, CH // 16, 16, 16), f32),
                  jax.ShapeDtypeStruct((NC, NP, ncls), f32)),
        mesh=mesh,
        compiler_params=pltpu.CompilerParams(needs_layout_passes=False, use_tc_tiling_on_sc=False),
        scratch_types=[
            pltpu.VMEM((NCHB, CH), i32),
            pltpu.VMEM((NCHB, CH), i32),
            pltpu.VMEM((NP,), f32),
            pltpu.VMEM((CH, ncls), f32),
            pltpu.VMEM((CH, ncls), f32),
            pltpu.VMEM((CH, ncls), f32),
            pltpu.VMEM((CH, ncls), f32),
            pltpu.VMEM((CH // 16, 16, 16), f32),
            pltpu.VMEM((CH // 16, 16, 16), f32),
            pltpu.VMEM((CH // 16, 16, 16), f32),
            pltpu.VMEM((CH // 16, 16, 16), f32),
            pltpu.VMEM((CH, ncls), f32),
            pltpu.VMEM((CH, ncls), f32),
            pltpu.VMEM((CH, ncls), f32),
            pltpu.VMEM((CH, ncls), f32),
            pltpu.VMEM((CH,), i32),
            pltpu.VMEM((CH,), i32),
            pltpu.VMEM((CH,), i32),
            pltpu.VMEM((CH,), i32),
            pltpu.VMEM((CH,), i32),
            pltpu.VMEM((CH,), i32),
            pltpu.VMEM((CH,), i32),
            pltpu.VMEM((CH,), i32),
            pltpu.VMEM_SHARED((NP, ncls), f32),
            pltpu.SemaphoreType.DMA,
            pltpu.SemaphoreType.DMA,
            pltpu.SemaphoreType.DMA,
            pltpu.SemaphoreType.DMA,
        ],
    )

    def _n2_body(y_hbm, t_hbm, xx_hbm, tc_hbm, yb, t0b, t1b, xxb, tcb):
        c = lax.axis_index("c")
        s = lax.axis_index("s")
        w = c * NS + s
        pltpu.sync_copy(y_hbm.at[pl.ds(w * RW, RW)], yb)
        pltpu.sync_copy(t_hbm.at[0, pl.ds(w * RW, RW)], t0b)
        pltpu.sync_copy(t_hbm.at[1, pl.ds(w * RW, RW)], t1b)

        def row(r, carry):
            tv = t0b[r, :] + t1b[r, :]
            tcb[r, :] = tv
            xxb[r, :] = yb[r, :] - GAMMA * tv
            return carry

        lax.fori_loop(0, RW, row, 0)
        pltpu.sync_copy(xxb, xx_hbm.at[pl.ds(w * RW, RW)])
        pltpu.sync_copy(tcb, tc_hbm.at[pl.ds(w * RW, RW)])

    n2k = pl.kernel(
        _n2_body,
        out_type=(jax.ShapeDtypeStruct((NP, ncls), f32),
                  jax.ShapeDtypeStruct((NP, ncls), f32)),
        mesh=mesh,
        compiler_params=pltpu.CompilerParams(needs_layout_passes=False, use_tc_tiling_on_sc=False),
        scratch_types=[pltpu.VMEM((RW, ncls), f32) for _ in range(5)],
    )

    xx = hp
    tcomb = z16
    zcur = z0
    y = hp
    acct = None
    for it in range(KITER):
        accp = propa(pr3, pc3, dinv, xx, z16)
        y, xbar = n1k(hp, accp, tcomb)
        zcur, acct = passb(se3, de3, dinv, xbar, zcur, z16)
        if it < KITER - 1:
            xx, tcomb = n2k(y, acct)

    fblk = 1280
    out = pl.pallas_call(
        _fin_body,
        grid=(NP // fblk,),
        in_specs=[
            pl.BlockSpec((fblk, ncls), lambda i: (i, 0)),
            pl.BlockSpec((fblk, ncls), lambda i: (i, 0)),
            pl.BlockSpec((fblk, ncls), lambda i: (i, 0)),
        ],
        out_specs=pl.BlockSpec((fblk, ncls), lambda i: (i, 0)),
        out_shape=jax.ShapeDtypeStruct((NP, ncls), f32),
    )(y, acct[0], acct[1])
    return out[:n]
